# baseline (speedup 1.0000x reference)
import jax
import jax.numpy as jnp
from jax.experimental import pallas as pl
from jax.experimental.pallas import tpu as pltpu

HIDDEN = 2048
B = 4 * 2048


def _body(idx_ref, in_ref, o_ref):
    o_ref[...] = in_ref[...]


def kernel(input_ids, wte):
    ids = input_ids.reshape(-1).astype(jnp.int32)
    wte3 = wte.reshape(-1, 1, HIDDEN)
    grid_spec = pltpu.PrefetchScalarGridSpec(
        num_scalar_prefetch=1,
        grid=(B,),
        in_specs=[pl.BlockSpec((1, 1, HIDDEN), lambda i, idx: (idx[i], 0, 0))],
        out_specs=pl.BlockSpec((1, 1, HIDDEN), lambda i, idx: (i, 0, 0)),
    )
    out = pl.pallas_call(
        _body, grid_spec=grid_spec,
        out_shape=jax.ShapeDtypeStruct((B, 1, HIDDEN), jnp.float32))(ids, wte3)
    return out.reshape(*input_ids.shape, HIDDEN)


# speedup vs baseline: 60.5814x; 60.5814x over previous
import functools

import jax
import jax.numpy as jnp
from jax import lax
from jax.experimental import pallas as pl
from jax.experimental.pallas import tpu as pltpu
from jax.experimental.pallas import tpu_sc as plsc

HIDDEN = 2048
B = 4 * 2048
NC, NS = 2, 16
NW = NC * NS
BPW = B // NW
CHUNK = 8
NCHUNK = BPW // CHUNK

_mesh = plsc.VectorSubcoreMesh(core_axis_name="c", subcore_axis_name="s")


NBUF = 6

_scratch = [pltpu.VMEM((NCHUNK, CHUNK), jnp.int32)]
_scratch += [pltpu.VMEM((CHUNK, HIDDEN), jnp.float32) for _ in range(NBUF)]
_scratch += [pltpu.SemaphoreType.DMA for _ in range(2 * NBUF)]


@functools.partial(
    pl.kernel,
    out_type=jax.ShapeDtypeStruct((B, HIDDEN), jnp.float32),
    mesh=_mesh,
    scratch_types=_scratch,
)
def _sc_gather(idx_hbm, table_hbm, out_hbm, idx_v, *bufs_and_sems):
    bufs = bufs_and_sems[:NBUF]
    gsems = bufs_and_sems[NBUF:2 * NBUF]
    wsems = bufs_and_sems[2 * NBUF:]

    wid = lax.axis_index("s") * NC + lax.axis_index("c")
    base = wid * BPW

    pltpu.sync_copy(idx_hbm.at[wid], idx_v)

    gather_h = [None] * NBUF
    write_h = [None] * NBUF

    for g in range(NBUF - 1):
        gather_h[g] = pltpu.async_copy(
            table_hbm.at[idx_v.at[g]], bufs[g], gsems[g])

    for g in range(NCHUNK):
        cur = g % NBUF
        nxt = (g + NBUF - 1) % NBUF
        if g + NBUF - 1 < NCHUNK:
            if write_h[nxt] is not None:
                write_h[nxt].wait()
                write_h[nxt] = None
            gather_h[nxt] = pltpu.async_copy(
                table_hbm.at[idx_v.at[g + NBUF - 1]], bufs[nxt], gsems[nxt])
        gather_h[cur].wait()
        write_h[cur] = pltpu.async_copy(
            bufs[cur], out_hbm.at[pl.ds(base + g * CHUNK, CHUNK)], wsems[cur])

    for h in write_h:
        if h is not None:
            h.wait()


def kernel(input_ids, wte):
    ids = input_ids.reshape(NW, NCHUNK, CHUNK).astype(jnp.int32)
    out = _sc_gather(ids, wte)
    return out.reshape(*input_ids.shape, HIDDEN)


# speedup vs baseline: 60.6594x; 1.0013x over previous
import functools

import jax
import jax.numpy as jnp
from jax import lax
from jax.experimental import pallas as pl
from jax.experimental.pallas import tpu as pltpu
from jax.experimental.pallas import tpu_sc as plsc

HIDDEN = 2048
BATCH = 4
SEQ = 2048
B = BATCH * SEQ
NC, NS = 2, 16
NW = NC * NS
BPW = B // NW
WPR = SEQ // BPW
CHUNK = 8
NCHUNK = BPW // CHUNK
NBUF = 6

_mesh = plsc.VectorSubcoreMesh(core_axis_name="c", subcore_axis_name="s")

_scratch = [pltpu.VMEM((BPW,), jnp.int32)]
_scratch += [pltpu.VMEM((CHUNK, HIDDEN), jnp.float32) for _ in range(NBUF)]
_scratch += [pltpu.SemaphoreType.DMA for _ in range(2 * NBUF)]


@functools.partial(
    pl.kernel,
    out_type=jax.ShapeDtypeStruct((BATCH, SEQ, HIDDEN), jnp.float32),
    mesh=_mesh,
    scratch_types=_scratch,
)
def _sc_gather(idx_hbm, table_hbm, out_hbm, idx_v, *bufs_and_sems):
    bufs = bufs_and_sems[:NBUF]
    gsems = bufs_and_sems[NBUF:2 * NBUF]
    wsems = bufs_and_sems[2 * NBUF:]

    wid = lax.axis_index("s") * NC + lax.axis_index("c")
    row = wid // WPR
    col = (wid % WPR) * BPW

    pltpu.sync_copy(idx_hbm.at[row, pl.ds(col, BPW)], idx_v)

    gather_h = [None] * NBUF
    write_h = [None] * NBUF

    for g in range(NBUF - 1):
        gather_h[g] = pltpu.async_copy(
            table_hbm.at[idx_v.at[pl.ds(g * CHUNK, CHUNK)]], bufs[g], gsems[g])

    for g in range(NCHUNK):
        cur = g % NBUF
        nxt = (g + NBUF - 1) % NBUF
        if g + NBUF - 1 < NCHUNK:
            if write_h[nxt] is not None:
                write_h[nxt].wait()
                write_h[nxt] = None
            gather_h[nxt] = pltpu.async_copy(
                table_hbm.at[idx_v.at[pl.ds((g + NBUF - 1) * CHUNK, CHUNK)]],
                bufs[nxt], gsems[nxt])
        gather_h[cur].wait()
        write_h[cur] = pltpu.async_copy(
            bufs[cur], out_hbm.at[row, pl.ds(col + g * CHUNK, CHUNK)],
            wsems[cur])

    for h in write_h:
        if h is not None:
            h.wait()


def kernel(input_ids, wte):
    return _sc_gather(input_ids.astype(jnp.int32), wte)


# speedup vs baseline: 60.6781x; 1.0003x over previous
import functools

import jax
import jax.numpy as jnp
from jax import lax
from jax.experimental import pallas as pl
from jax.experimental.pallas import tpu as pltpu
from jax.experimental.pallas import tpu_sc as plsc

HIDDEN = 2048
BATCH = 4
SEQ = 2048
B = BATCH * SEQ
NC, NS = 2, 16
NW = NC * NS
BPW = B // NW
WPR = SEQ // BPW
CHUNK = 8
NCHUNK = BPW // CHUNK
NBUF = 7

_mesh = plsc.VectorSubcoreMesh(core_axis_name="c", subcore_axis_name="s")

_scratch = [pltpu.VMEM((BPW,), jnp.int32)]
_scratch += [pltpu.VMEM((CHUNK, HIDDEN), jnp.float32) for _ in range(NBUF)]
_scratch += [pltpu.SemaphoreType.DMA for _ in range(2 * NBUF)]


@functools.partial(
    pl.kernel,
    out_type=jax.ShapeDtypeStruct((BATCH, SEQ, HIDDEN), jnp.float32),
    mesh=_mesh,
    scratch_types=_scratch,
)
def _sc_gather(idx_hbm, table_hbm, out_hbm, idx_v, *bufs_and_sems):
    bufs = bufs_and_sems[:NBUF]
    gsems = bufs_and_sems[NBUF:2 * NBUF]
    wsems = bufs_and_sems[2 * NBUF:]

    wid = lax.axis_index("s") * NC + lax.axis_index("c")
    row = wid // WPR
    col = (wid % WPR) * BPW

    pltpu.sync_copy(idx_hbm.at[row, pl.ds(col, BPW)], idx_v)

    gather_h = [None] * NBUF
    write_h = [None] * NBUF

    for g in range(NBUF - 1):
        gather_h[g] = pltpu.async_copy(
            table_hbm.at[idx_v.at[pl.ds(g * CHUNK, CHUNK)]], bufs[g], gsems[g])

    for g in range(NCHUNK):
        cur = g % NBUF
        nxt = (g + NBUF - 1) % NBUF
        if g + NBUF - 1 < NCHUNK:
            if write_h[nxt] is not None:
                write_h[nxt].wait()
                write_h[nxt] = None
            gather_h[nxt] = pltpu.async_copy(
                table_hbm.at[idx_v.at[pl.ds((g + NBUF - 1) * CHUNK, CHUNK)]],
                bufs[nxt], gsems[nxt])
        gather_h[cur].wait()
        write_h[cur] = pltpu.async_copy(
            bufs[cur], out_hbm.at[row, pl.ds(col + g * CHUNK, CHUNK)],
            wsems[cur])

    for h in write_h:
        if h is not None:
            h.wait()


def kernel(input_ids, wte):
    return _sc_gather(input_ids.astype(jnp.int32), wte)


# speedup vs baseline: 60.7328x; 1.0009x over previous
import functools

import jax
import jax.numpy as jnp
from jax import lax
from jax.experimental import pallas as pl
from jax.experimental.pallas import tpu as pltpu
from jax.experimental.pallas import tpu_sc as plsc

HIDDEN = 2048
BATCH = 4
SEQ = 2048
B = BATCH * SEQ
NC, NS = 2, 16
NW = NC * NS
BPW = B // NW
WPR = SEQ // BPW
CHUNK = 8
NCHUNK = BPW // CHUNK
NBUF = 6

_mesh = plsc.VectorSubcoreMesh(core_axis_name="c", subcore_axis_name="s")

_scratch = [pltpu.VMEM((BPW,), jnp.int32)]
_scratch += [pltpu.VMEM((CHUNK, HIDDEN), jnp.float32) for _ in range(NBUF)]
_scratch += [pltpu.SemaphoreType.DMA for _ in range(2 * NBUF)]


@functools.partial(
    pl.kernel,
    out_type=jax.ShapeDtypeStruct((BATCH, SEQ, HIDDEN), jnp.float32),
    mesh=_mesh,
    scratch_types=_scratch,
)
def _sc_gather(idx_hbm, table_hbm, out_hbm, idx_v, *bufs_and_sems):
    bufs = bufs_and_sems[:NBUF]
    gsems = bufs_and_sems[NBUF:2 * NBUF]
    wsems = bufs_and_sems[2 * NBUF:]

    wid = lax.axis_index("s") * NC + lax.axis_index("c")
    row = wid // WPR
    col = (wid % WPR) * BPW

    pltpu.sync_copy(idx_hbm.at[row, pl.ds(col, BPW)], idx_v)

    gather_h = [None] * NBUF
    write_h = [None] * NBUF

    for g in range(NBUF - 1):
        gather_h[g] = pltpu.async_copy(
            table_hbm.at[idx_v.at[pl.ds(g * CHUNK, CHUNK)]], bufs[g], gsems[g])

    for g in range(NCHUNK):
        cur = g % NBUF
        nxt = (g + NBUF - 1) % NBUF
        if g + NBUF - 1 < NCHUNK:
            if write_h[nxt] is not None:
                write_h[nxt].wait()
                write_h[nxt] = None
            gather_h[nxt] = pltpu.async_copy(
                table_hbm.at[idx_v.at[pl.ds((g + NBUF - 1) * CHUNK, CHUNK)]],
                bufs[nxt], gsems[nxt])
        gather_h[cur].wait()
        write_h[cur] = pltpu.async_copy(
            bufs[cur], out_hbm.at[row, pl.ds(col + g * CHUNK, CHUNK)],
            wsems[cur])

    for h in write_h:
        if h is not None:
            h.wait()


def kernel(input_ids, wte):
    return _sc_gather(input_ids.astype(jnp.int32), wte)
